# Initial kernel scaffold; baseline (speedup 1.0000x reference)
import jax
import jax.numpy as jnp
from jax.experimental import pallas as pl


def kernel(x, edge_index, W1, b1, W2, b2):
    raise NotImplementedError("write your pallas kernel here")


# speedup vs baseline: 6.0062x; 6.0062x over previous
import functools

import jax
import jax.numpy as jnp
from jax import lax
from jax.experimental import pallas as pl
from jax.experimental.pallas import tpu as pltpu
from jax.experimental.pallas import tpu_sc as plsc

N = 10000
NP = 10240
E = 320000
NW = 32
CH = 80
K = 128
EP = NW * CH * K
HID = 128
LAT = 64
ZR = 32
BM = 1280
GRID = NP // BM

HALF = NP // 2
HROWS = HALF // 16
CH2 = 2 * CH


@functools.lru_cache(maxsize=None)
def _mesh():
    return plsc.VectorSubcoreMesh(core_axis_name="c", subcore_axis_name="s")


def _deg_body(dstr, out, didx, ones_v, zbuf, acc):
    c = lax.axis_index("c")
    s = lax.axis_index("s")
    wid = s * 2 + c
    pltpu.sync_copy(dstr.at[wid], didx)
    z16 = jnp.zeros((16,), jnp.float32)
    o16 = jnp.ones((16,), jnp.float32)
    for j in range(K // 16):
        ones_v[pl.ds(16 * j, 16)] = o16
    rpt = NP // 16
    for j in range(rpt // 16):
        zbuf[pl.ds(16 * j, 16)] = z16
    base = s * rpt
    pltpu.sync_copy(zbuf, acc.at[pl.ds(base, rpt)])
    plsc.subcore_barrier()

    def body(i, carry):
        pltpu.sync_copy(ones_v, acc.at[didx.at[i]], add=True)
        return carry

    lax.fori_loop(0, CH, body, 0)
    plsc.subcore_barrier()
    pltpu.sync_copy(acc.at[pl.ds(base, rpt)], out.at[c, pl.ds(base, rpt)])


@functools.lru_cache(maxsize=None)
def _deg_call():
    return pl.kernel(
        _deg_body,
        out_type=jax.ShapeDtypeStruct((2, NP), jnp.float32),
        mesh=_mesh(),
        scratch_types=[
            pltpu.VMEM((CH, K), jnp.int32),
            pltpu.VMEM((K,), jnp.float32),
            pltpu.VMEM((NP // 16,), jnp.float32),
            pltpu.VMEM_SHARED((NP,), jnp.float32),
        ],
    )


def _agg_body(d, hp, srcr, dstr, out, sidx, didx, dloc, rows, zbuf, acc,
              sem0, sem1):
    c = lax.axis_index("c")
    s = lax.axis_index("s")
    pltpu.sync_copy(srcr.at[2 * s], sidx.at[pl.ds(0, CH)])
    pltpu.sync_copy(srcr.at[2 * s + 1], sidx.at[pl.ds(CH, CH)])
    pltpu.sync_copy(dstr.at[2 * s], didx.at[pl.ds(0, CH)])
    pltpu.sync_copy(dstr.at[2 * s + 1], didx.at[pl.ds(CH, CH)])
    z16 = jnp.zeros((16,), jnp.float32)
    for i in range(ZR):
        for j in range(d // 16):
            zbuf[i, pl.ds(16 * j, 16)] = z16
    base = s * HROWS

    def zloop(j, carry):
        pltpu.sync_copy(zbuf, acc.at[pl.ds(base + j * ZR, ZR)])
        return carry

    lax.fori_loop(0, HROWS // ZR, zloop, 0)
    plsc.subcore_barrier()

    lhalf = c * HALF

    def localize(ci, buf):
        for j in range(K // 16):
            v = didx[ci, pl.ds(16 * j, 16)]
            vl = v - lhalf
            m = (vl >= 0) & (vl < HALF)
            dloc[buf, pl.ds(16 * j, 16)] = jnp.where(m, vl, HALF)

    ng = CH2 // 2
    pltpu.async_copy(hp.at[sidx.at[0]], rows.at[0], sem0)

    def body(g, carry):
        c0 = 2 * g
        c1 = c0 + 1
        localize(c0, 0)
        pltpu.make_async_copy(hp.at[sidx.at[c0]], rows.at[0], sem0).wait()
        pltpu.async_copy(hp.at[sidx.at[c1]], rows.at[1], sem1)
        pltpu.sync_copy(rows.at[0], acc.at[dloc.at[0]], add=True)
        localize(c1, 1)
        pltpu.make_async_copy(hp.at[sidx.at[c1]], rows.at[1], sem1).wait()

        @pl.when(g + 1 < ng)
        def _():
            pltpu.async_copy(hp.at[sidx.at[c0 + 2]], rows.at[0], sem0)

        pltpu.sync_copy(rows.at[1], acc.at[dloc.at[1]], add=True)
        return carry

    lax.fori_loop(0, ng, body, 0)
    plsc.subcore_barrier()
    pltpu.sync_copy(acc.at[pl.ds(base, HROWS)],
                    out.at[pl.ds(lhalf + base, HROWS)])


@functools.lru_cache(maxsize=None)
def _agg_call(d):
    return pl.kernel(
        functools.partial(_agg_body, d),
        out_type=jax.ShapeDtypeStruct((NP, d), jnp.float32),
        mesh=_mesh(),
        scratch_types=[
            pltpu.VMEM((CH2, K), jnp.int32),
            pltpu.VMEM((CH2, K), jnp.int32),
            pltpu.VMEM((2, K), jnp.int32),
            pltpu.VMEM((2, K, d), jnp.float32),
            pltpu.VMEM((ZR, d), jnp.float32),
            pltpu.VMEM_SHARED((HALF + 8, d), jnp.float32),
            pltpu.SemaphoreType.DMA,
            pltpu.SemaphoreType.DMA,
        ],
    )


def _tc1_body(x_ref, w_ref, deg_ref, o_ref):
    dinv = lax.rsqrt(deg_ref[...] + 1.0)
    o_ref[...] = jnp.dot(x_ref[...], w_ref[...],
                         preferred_element_type=jnp.float32) * dinv


def _tc2_body(p_ref, h_ref, deg_ref, b_ref, o_ref):
    dinv = lax.rsqrt(deg_ref[...] + 1.0)
    z = (p_ref[...] + h_ref[...]) * dinv + b_ref[...]
    o_ref[...] = jnp.maximum(z, 0.0) * dinv


def _tc3_body(p_ref, h_ref, deg_ref, b_ref, w_ref, o_ref):
    dinv = lax.rsqrt(deg_ref[...] + 1.0)
    z = jnp.dot(p_ref[...] + h_ref[...], w_ref[...],
                preferred_element_type=jnp.float32) * dinv + b_ref[...]
    o_ref[...] = jnp.maximum(z, 0.0)


def _tc1(x_p, W1, deg2):
    return pl.pallas_call(
        _tc1_body,
        grid=(GRID,),
        in_specs=[
            pl.BlockSpec((BM, HID), lambda i: (i, 0)),
            pl.BlockSpec((HID, HID), lambda i: (0, 0)),
            pl.BlockSpec((BM, 1), lambda i: (i, 0)),
        ],
        out_specs=pl.BlockSpec((BM, HID), lambda i: (i, 0)),
        out_shape=jax.ShapeDtypeStruct((NP, HID), jnp.float32),
    )(x_p, W1, deg2)


def _tc2(p1, h1p, deg2, b1):
    return pl.pallas_call(
        _tc2_body,
        grid=(GRID,),
        in_specs=[
            pl.BlockSpec((BM, HID), lambda i: (i, 0)),
            pl.BlockSpec((BM, HID), lambda i: (i, 0)),
            pl.BlockSpec((BM, 1), lambda i: (i, 0)),
            pl.BlockSpec((1, HID), lambda i: (0, 0)),
        ],
        out_specs=pl.BlockSpec((BM, HID), lambda i: (i, 0)),
        out_shape=jax.ShapeDtypeStruct((NP, HID), jnp.float32),
    )(p1, h1p, deg2, b1)


def _tc3(p2, z1p, deg2, b2, W2):
    return pl.pallas_call(
        _tc3_body,
        grid=(GRID,),
        in_specs=[
            pl.BlockSpec((BM, HID), lambda i: (i, 0)),
            pl.BlockSpec((BM, HID), lambda i: (i, 0)),
            pl.BlockSpec((BM, 1), lambda i: (i, 0)),
            pl.BlockSpec((1, LAT), lambda i: (0, 0)),
            pl.BlockSpec((HID, LAT), lambda i: (0, 0)),
        ],
        out_specs=pl.BlockSpec((BM, LAT), lambda i: (i, 0)),
        out_shape=jax.ShapeDtypeStruct((NP, LAT), jnp.float32),
    )(p2, z1p, deg2, b2, W2)


def kernel(x, edge_index, W1, b1, W2, b2):
    ei = edge_index.astype(jnp.int32)
    pad_idx = jnp.full((EP - E,), N, jnp.int32)
    src_p = jnp.concatenate([ei[0], pad_idx]).reshape(NW, CH, K)
    dst_p = jnp.concatenate([ei[1], pad_idx]).reshape(NW, CH, K)
    x_p = jnp.pad(x, ((0, NP - N), (0, 0)))

    degp = _deg_call()(dst_p)
    deg2 = (degp[0] + degp[1]).reshape(NP, 1)
    h1p = _tc1(x_p, W1, deg2)
    p1 = _agg_call(HID)(h1p, src_p, dst_p)
    z1p = _tc2(p1, h1p, deg2, b1.reshape(1, HID))
    p2 = _agg_call(HID)(z1p, src_p, dst_p)
    outp = _tc3(p2, z1p, deg2, b2.reshape(1, LAT), W2)
    return outp[:N]
